# Initial kernel scaffold; baseline (speedup 1.0000x reference)
import jax
import jax.numpy as jnp
from jax.experimental import pallas as pl


def kernel(importance, training):
    raise NotImplementedError("write your pallas kernel here")


# speedup vs baseline: 97.0432x; 97.0432x over previous
import functools

import jax
import jax.numpy as jnp
from jax.experimental import pallas as pl
from jax.experimental.pallas import tpu as pltpu

_TARGET_RATE = 0.75


def _mask_body(k, x_ref, mask_ref, cnt_ref):
    x = x_ref[0]
    bits = jax.lax.bitcast_convert_type(x, jnp.int32)
    n = x.size

    def step(_, carry):
        lo, hi, c_lo = carry
        mid = (lo + hi) >> 1
        c = jnp.sum((bits >= mid).astype(jnp.int32))
        take = c >= k
        lo = jnp.where(take, mid, lo)
        hi = jnp.where(take, hi, mid)
        c_lo = jnp.where(take, c, c_lo)
        return lo, hi, c_lo

    lo0 = jnp.int32(0)
    hi0 = jnp.int32(0x3F800000)
    thr, _, c_final = jax.lax.fori_loop(0, 30, step, (lo0, hi0, jnp.int32(n)))

    mask_ref[0] = (bits >= thr).astype(jnp.float32)
    cnt_ref[0, 0, 0] = c_final


def kernel(importance, training):
    del training
    B, _, H, W = importance.shape
    n = H * W
    k = max(1, int(_TARGET_RATE * n))
    x = importance.reshape(B, H, W)

    mask, counts = pl.pallas_call(
        functools.partial(_mask_body, k),
        grid=(B,),
        in_specs=[pl.BlockSpec((1, H, W), lambda b: (b, 0, 0))],
        out_specs=[
            pl.BlockSpec((1, H, W), lambda b: (b, 0, 0)),
            pl.BlockSpec((1, 1, 1), lambda b: (b, 0, 0), memory_space=pltpu.SMEM),
        ],
        out_shape=[
            jax.ShapeDtypeStruct((B, H, W), jnp.float32),
            jax.ShapeDtypeStruct((B, 1, 1), jnp.int32),
        ],
    )(x)

    mean = jnp.sum(counts).astype(jnp.float32) / jnp.float32(B * n)
    return (mask[:, None, :, :], mean)
